# Initial kernel scaffold; baseline (speedup 1.0000x reference)
import jax
import jax.numpy as jnp
from jax.experimental import pallas as pl


def kernel(x, edge_index, edge_attr, eW1, eb1, eW2, eb2, nW1, nb1, nW2, nb2):
    raise NotImplementedError("write your pallas kernel here")


# speedup vs baseline: 3.1941x; 3.1941x over previous
import functools

import jax
import jax.numpy as jnp
from jax import lax
from jax.experimental import pallas as pl
from jax.experimental.pallas import tpu as pltpu
from jax.experimental.pallas import tpu_sc as plsc

LAT = 128
N_NODES_K = 10000
N_EDGES_K = 320000

NC, NS = 2, 16
NW = NC * NS
CHUNK = 128
N_CHUNKS = N_EDGES_K // CHUNK
CPW = (N_CHUNKS + NW - 1) // NW
NFULL = N_NODES_K // CHUNK
NTAIL = N_NODES_K - NFULL * CHUNK
NSLICE = NFULL + 1
SPS = (NSLICE + NS - 1) // NS

_SC_MESH = plsc.VectorSubcoreMesh(
    core_axis_name="c", subcore_axis_name="s", num_cores=NC, num_subcores=NS)


def _pre_body(x_ref, wa_ref, wb_ref, b1_ref, xa_ref, xb_ref):
    x = x_ref[...]
    xa_ref[...] = jnp.dot(x, wa_ref[...], preferred_element_type=jnp.float32)
    xb_ref[...] = (jnp.dot(x, wb_ref[...], preferred_element_type=jnp.float32)
                   + b1_ref[...])


_pre_call = pl.pallas_call(
    _pre_body,
    out_shape=(jax.ShapeDtypeStruct((N_NODES_K, LAT), jnp.float32),
               jax.ShapeDtypeStruct((N_NODES_K, LAT), jnp.float32)),
)


def _gather_body(xa_hbm, xb_hbm, row_hbm, col_hbm, ga_hbm, gb_hbm,
                 idx_r, idx_c, buf_a, buf_b, sem_a, sem_b):
    wid = lax.axis_index("s") * NC + lax.axis_index("c")

    def chunk(i, carry):
        c = wid * CPW + i

        @pl.when(c < N_CHUNKS)
        def _():
            base = c * CHUNK
            pltpu.sync_copy(row_hbm.at[pl.ds(base, CHUNK)], idx_r)
            pltpu.sync_copy(col_hbm.at[pl.ds(base, CHUNK)], idx_c)
            cp_a = pltpu.async_copy(xa_hbm.at[idx_r], buf_a, sem_a)
            cp_b = pltpu.async_copy(xb_hbm.at[idx_c], buf_b, sem_b)
            cp_a.wait()
            cp_b.wait()
            pltpu.sync_copy(buf_a, ga_hbm.at[pl.ds(base, CHUNK)])
            pltpu.sync_copy(buf_b, gb_hbm.at[pl.ds(base, CHUNK)])

        return carry

    lax.fori_loop(0, CPW, chunk, 0)


_gather_call = pl.kernel(
    _gather_body,
    out_type=(jax.ShapeDtypeStruct((N_EDGES_K, LAT), jnp.float32),
              jax.ShapeDtypeStruct((N_EDGES_K, LAT), jnp.float32)),
    mesh=_SC_MESH,
    scratch_types=(
        pltpu.VMEM((CHUNK,), jnp.int32),
        pltpu.VMEM((CHUNK,), jnp.int32),
        pltpu.VMEM((CHUNK, LAT), jnp.float32),
        pltpu.VMEM((CHUNK, LAT), jnp.float32),
        pltpu.SemaphoreType.DMA,
        pltpu.SemaphoreType.DMA,
    ),
)


EBLK = 4000


def _edge_body(ga_ref, gb_ref, e_ref, w1c_ref, w2_ref, b2_ref, out_ref):
    e = e_ref[...]
    h = jnp.dot(e, w1c_ref[...], preferred_element_type=jnp.float32)
    h = jnp.maximum(h + ga_ref[...] + gb_ref[...], 0.0)
    out_ref[...] = (e + jnp.dot(h, w2_ref[...],
                                preferred_element_type=jnp.float32)
                    + b2_ref[...])


_edge_call = pl.pallas_call(
    _edge_body,
    grid=(N_EDGES_K // EBLK,),
    in_specs=[
        pl.BlockSpec((EBLK, LAT), lambda i: (i, 0)),
        pl.BlockSpec((EBLK, LAT), lambda i: (i, 0)),
        pl.BlockSpec((EBLK, LAT), lambda i: (i, 0)),
        pl.BlockSpec((LAT, LAT), lambda i: (0, 0)),
        pl.BlockSpec((LAT, LAT), lambda i: (0, 0)),
        pl.BlockSpec((1, LAT), lambda i: (0, 0)),
    ],
    out_specs=pl.BlockSpec((EBLK, LAT), lambda i: (i, 0)),
    out_shape=jax.ShapeDtypeStruct((N_EDGES_K, LAT), jnp.float32),
    compiler_params=pltpu.CompilerParams(
        dimension_semantics=("parallel",)),
)


def _scatter_body(en_hbm, col_hbm, out_hbm, acc_shared, idx_v, buf, zbuf):
    cid = lax.axis_index("c")
    sid = lax.axis_index("s")
    wid = sid * NC + cid

    def zrow(r, carry):
        for j in range(LAT // 16):
            zbuf[r, pl.ds(j * 16, 16)] = jnp.zeros((16,), jnp.float32)
        return carry

    lax.fori_loop(0, CHUNK, zrow, 0)

    def zslice(i, carry):
        j = i * NS + sid

        @pl.when(j < NFULL)
        def _():
            pltpu.sync_copy(zbuf, acc_shared.at[pl.ds(j * CHUNK, CHUNK)])

        @pl.when(j == NFULL)
        def _():
            pltpu.sync_copy(zbuf.at[pl.ds(0, NTAIL)],
                            acc_shared.at[pl.ds(NFULL * CHUNK, NTAIL)])

        return carry

    lax.fori_loop(0, SPS, zslice, 0)
    plsc.subcore_barrier()

    def chunk(i, carry):
        c = wid * CPW + i

        @pl.when(c < N_CHUNKS)
        def _():
            base = c * CHUNK
            pltpu.sync_copy(col_hbm.at[pl.ds(base, CHUNK)], idx_v)
            pltpu.sync_copy(en_hbm.at[pl.ds(base, CHUNK)], buf)
            pltpu.sync_copy(buf, acc_shared.at[idx_v], add=True)

        return carry

    lax.fori_loop(0, CPW, chunk, 0)
    plsc.subcore_barrier()

    def oslice(i, carry):
        j = i * NS + sid

        @pl.when(j < NFULL)
        def _():
            pltpu.sync_copy(acc_shared.at[pl.ds(j * CHUNK, CHUNK)],
                            out_hbm.at[cid, pl.ds(j * CHUNK, CHUNK)])

        @pl.when(j == NFULL)
        def _():
            pltpu.sync_copy(acc_shared.at[pl.ds(NFULL * CHUNK, NTAIL)],
                            out_hbm.at[cid, pl.ds(NFULL * CHUNK, NTAIL)])

        return carry

    lax.fori_loop(0, SPS, oslice, 0)


_scatter_call = pl.kernel(
    _scatter_body,
    out_type=jax.ShapeDtypeStruct((NC, N_NODES_K, LAT), jnp.float32),
    mesh=_SC_MESH,
    scratch_types=(
        pltpu.VMEM_SHARED((N_NODES_K, LAT), jnp.float32),
        pltpu.VMEM((CHUNK,), jnp.int32),
        pltpu.VMEM((CHUNK, LAT), jnp.float32),
        pltpu.VMEM((CHUNK, LAT), jnp.float32),
    ),
)


def _node_body(x_ref, agg_ref, w1a_ref, w1b_ref, b1_ref, w2_ref, b2_ref,
               out_ref):
    x = x_ref[...]
    s = agg_ref[0] + agg_ref[1]
    h = (jnp.dot(x, w1a_ref[...], preferred_element_type=jnp.float32)
         + jnp.dot(s, w1b_ref[...], preferred_element_type=jnp.float32)
         + b1_ref[...])
    h = jnp.maximum(h, 0.0)
    out_ref[...] = (x + jnp.dot(h, w2_ref[...],
                                preferred_element_type=jnp.float32)
                    + b2_ref[...])


_node_call = pl.pallas_call(
    _node_body,
    out_shape=jax.ShapeDtypeStruct((N_NODES_K, LAT), jnp.float32),
)


def kernel(x, edge_index, edge_attr, eW1, eb1, eW2, eb2, nW1, nb1, nW2, nb2):
    row = edge_index[0]
    col = edge_index[1]
    for l in range(2):
        w1a = eW1[l, :LAT]
        w1b = eW1[l, LAT:2 * LAT]
        w1c = eW1[l, 2 * LAT:]
        xa, xb = _pre_call(x, w1a, w1b, eb1[l].reshape(1, LAT))
        ga, gb = _gather_call(xa, xb, row, col)
        edge_attr = _edge_call(ga, gb, edge_attr, w1c, eW2[l],
                               eb2[l].reshape(1, LAT))
        agg2 = _scatter_call(edge_attr, col)
        x = _node_call(x, agg2, nW1[l, :LAT], nW1[l, LAT:],
                       nb1[l].reshape(1, LAT), nW2[l],
                       nb2[l].reshape(1, LAT))
    return (x, edge_attr)


# speedup vs baseline: 3.6016x; 1.1276x over previous
import functools

import jax
import jax.numpy as jnp
from jax import lax
from jax.experimental import pallas as pl
from jax.experimental.pallas import tpu as pltpu
from jax.experimental.pallas import tpu_sc as plsc

LAT = 128
N_NODES_K = 10000
N_EDGES_K = 320000

NC, NS = 2, 16
NW = NC * NS
CHUNK = 128
N_CHUNKS = N_EDGES_K // CHUNK
CPW = (N_CHUNKS + NW - 1) // NW
NFULL = N_NODES_K // CHUNK
NTAIL = N_NODES_K - NFULL * CHUNK
NSLICE = NFULL + 1
SPS = (NSLICE + NS - 1) // NS

_SC_MESH = plsc.VectorSubcoreMesh(
    core_axis_name="c", subcore_axis_name="s", num_cores=NC, num_subcores=NS)


def _pre_body(x_ref, wa_ref, wb_ref, b1_ref, xa_ref, xb_ref):
    x = x_ref[...]
    xa_ref[...] = jnp.dot(x, wa_ref[...], preferred_element_type=jnp.float32)
    xb_ref[...] = (jnp.dot(x, wb_ref[...], preferred_element_type=jnp.float32)
                   + b1_ref[...])


_pre_call = pl.pallas_call(
    _pre_body,
    out_shape=(jax.ShapeDtypeStruct((N_NODES_K, LAT), jnp.float32),
               jax.ShapeDtypeStruct((N_NODES_K, LAT), jnp.float32)),
)


LROWS = 640
LTAIL = N_NODES_K - 15 * LROWS


def _gather_body(xa_hbm, xb_hbm, row_hbm, col_hbm, g_hbm,
                 xa_sh, idx_r, idx_c, buf):
    sid = lax.axis_index("s")
    wid = sid * NC + lax.axis_index("c")

    @pl.when(sid < NS - 1)
    def _():
        pltpu.sync_copy(xa_hbm.at[pl.ds(sid * LROWS, LROWS)],
                        xa_sh.at[pl.ds(sid * LROWS, LROWS)])

    @pl.when(sid == NS - 1)
    def _():
        pltpu.sync_copy(xa_hbm.at[pl.ds((NS - 1) * LROWS, LTAIL)],
                        xa_sh.at[pl.ds((NS - 1) * LROWS, LTAIL)])

    plsc.subcore_barrier()

    def chunk(i, carry):
        c = wid * CPW + i

        @pl.when(c < N_CHUNKS)
        def _():
            base = c * CHUNK
            pltpu.sync_copy(row_hbm.at[pl.ds(base, CHUNK)], idx_r)
            pltpu.sync_copy(col_hbm.at[pl.ds(base, CHUNK)], idx_c)
            pltpu.sync_copy(xa_sh.at[idx_r], buf)
            pltpu.sync_copy(xb_hbm.at[idx_c], buf, add=True)
            pltpu.sync_copy(buf, g_hbm.at[pl.ds(base, CHUNK)])

        return carry

    lax.fori_loop(0, CPW, chunk, 0)


_gather_call = pl.kernel(
    _gather_body,
    out_type=jax.ShapeDtypeStruct((N_EDGES_K, LAT), jnp.float32),
    mesh=_SC_MESH,
    scratch_types=(
        pltpu.VMEM_SHARED((N_NODES_K, LAT), jnp.float32),
        pltpu.VMEM((CHUNK,), jnp.int32),
        pltpu.VMEM((CHUNK,), jnp.int32),
        pltpu.VMEM((CHUNK, LAT), jnp.float32),
    ),
)


EBLK = 4000


def _edge_body(g_ref, e_ref, w1c_ref, w2_ref, b2_ref, out_ref):
    e = e_ref[...]
    h = jnp.dot(e.astype(jnp.bfloat16), w1c_ref[...].astype(jnp.bfloat16),
                preferred_element_type=jnp.float32)
    h = jnp.maximum(h + g_ref[...].astype(jnp.float32), 0.0)
    out_ref[...] = (e + jnp.dot(h.astype(jnp.bfloat16),
                                w2_ref[...].astype(jnp.bfloat16),
                                preferred_element_type=jnp.float32)
                    + b2_ref[...])


_edge_call = pl.pallas_call(
    _edge_body,
    grid=(N_EDGES_K // EBLK,),
    in_specs=[
        pl.BlockSpec((EBLK, LAT), lambda i: (i, 0)),
        pl.BlockSpec((EBLK, LAT), lambda i: (i, 0)),
        pl.BlockSpec((LAT, LAT), lambda i: (0, 0)),
        pl.BlockSpec((LAT, LAT), lambda i: (0, 0)),
        pl.BlockSpec((1, LAT), lambda i: (0, 0)),
    ],
    out_specs=pl.BlockSpec((EBLK, LAT), lambda i: (i, 0)),
    out_shape=jax.ShapeDtypeStruct((N_EDGES_K, LAT), jnp.float32),
    compiler_params=pltpu.CompilerParams(
        dimension_semantics=("parallel",)),
)


def _scatter_body(en_hbm, col_hbm, out_hbm, acc_shared, idx_v, buf, zbuf):
    cid = lax.axis_index("c")
    sid = lax.axis_index("s")
    wid = sid * NC + cid

    def zrow(r, carry):
        for j in range(LAT // 16):
            zbuf[r, pl.ds(j * 16, 16)] = jnp.zeros((16,), jnp.float32)
        return carry

    lax.fori_loop(0, CHUNK, zrow, 0)

    def zslice(i, carry):
        j = i * NS + sid

        @pl.when(j < NFULL)
        def _():
            pltpu.sync_copy(zbuf, acc_shared.at[pl.ds(j * CHUNK, CHUNK)])

        @pl.when(j == NFULL)
        def _():
            pltpu.sync_copy(zbuf.at[pl.ds(0, NTAIL)],
                            acc_shared.at[pl.ds(NFULL * CHUNK, NTAIL)])

        return carry

    lax.fori_loop(0, SPS, zslice, 0)
    plsc.subcore_barrier()

    def chunk(i, carry):
        c = wid * CPW + i

        @pl.when(c < N_CHUNKS)
        def _():
            base = c * CHUNK
            pltpu.sync_copy(col_hbm.at[pl.ds(base, CHUNK)], idx_v)
            pltpu.sync_copy(en_hbm.at[pl.ds(base, CHUNK)], buf)
            pltpu.sync_copy(buf, acc_shared.at[idx_v], add=True)

        return carry

    lax.fori_loop(0, CPW, chunk, 0)
    plsc.subcore_barrier()

    def oslice(i, carry):
        j = i * NS + sid

        @pl.when(j < NFULL)
        def _():
            pltpu.sync_copy(acc_shared.at[pl.ds(j * CHUNK, CHUNK)],
                            out_hbm.at[cid, pl.ds(j * CHUNK, CHUNK)])

        @pl.when(j == NFULL)
        def _():
            pltpu.sync_copy(acc_shared.at[pl.ds(NFULL * CHUNK, NTAIL)],
                            out_hbm.at[cid, pl.ds(NFULL * CHUNK, NTAIL)])

        return carry

    lax.fori_loop(0, SPS, oslice, 0)


_scatter_call = pl.kernel(
    _scatter_body,
    out_type=jax.ShapeDtypeStruct((NC, N_NODES_K, LAT), jnp.float32),
    mesh=_SC_MESH,
    scratch_types=(
        pltpu.VMEM_SHARED((N_NODES_K, LAT), jnp.float32),
        pltpu.VMEM((CHUNK,), jnp.int32),
        pltpu.VMEM((CHUNK, LAT), jnp.float32),
        pltpu.VMEM((CHUNK, LAT), jnp.float32),
    ),
)


def _node_body(x_ref, agg_ref, w1a_ref, w1b_ref, b1_ref, w2_ref, b2_ref,
               out_ref):
    x = x_ref[...]
    s = agg_ref[0] + agg_ref[1]
    h = (jnp.dot(x, w1a_ref[...], preferred_element_type=jnp.float32)
         + jnp.dot(s, w1b_ref[...], preferred_element_type=jnp.float32)
         + b1_ref[...])
    h = jnp.maximum(h, 0.0)
    out_ref[...] = (x + jnp.dot(h, w2_ref[...],
                                preferred_element_type=jnp.float32)
                    + b2_ref[...])


_node_call = pl.pallas_call(
    _node_body,
    out_shape=jax.ShapeDtypeStruct((N_NODES_K, LAT), jnp.float32),
)


def kernel(x, edge_index, edge_attr, eW1, eb1, eW2, eb2, nW1, nb1, nW2, nb2):
    row = edge_index[0]
    col = edge_index[1]
    for l in range(2):
        w1a = eW1[l, :LAT]
        w1b = eW1[l, LAT:2 * LAT]
        w1c = eW1[l, 2 * LAT:]
        xa, xb = _pre_call(x, w1a, w1b, eb1[l].reshape(1, LAT))
        g = _gather_call(xa, xb, row, col)
        edge_attr = _edge_call(g, edge_attr, w1c, eW2[l],
                               eb2[l].reshape(1, LAT))
        agg2 = _scatter_call(edge_attr, col)
        x = _node_call(x, agg2, nW1[l, :LAT], nW1[l, LAT:],
                       nb1[l].reshape(1, LAT), nW2[l],
                       nb2[l].reshape(1, LAT))
    return (x, edge_attr)


# speedup vs baseline: 4.9888x; 1.3852x over previous
import functools

import jax
import jax.numpy as jnp
from jax import lax
from jax.experimental import pallas as pl
from jax.experimental.pallas import tpu as pltpu
from jax.experimental.pallas import tpu_sc as plsc

LAT = 128
N_NODES_K = 10000
N_EDGES_K = 320000

NC, NS = 2, 16
NW = NC * NS
CHUNK = 128
N_CHUNKS = N_EDGES_K // CHUNK
CPW = (N_CHUNKS + NW - 1) // NW
NFULL = N_NODES_K // CHUNK
NTAIL = N_NODES_K - NFULL * CHUNK
NSLICE = NFULL + 1
SPS = (NSLICE + NS - 1) // NS
ZROWS = 16
NZ = N_NODES_K // ZROWS
ZPS = (NZ + NS - 1) // NS

_SC_MESH = plsc.VectorSubcoreMesh(
    core_axis_name="c", subcore_axis_name="s", num_cores=NC, num_subcores=NS)


def _pre_body(x_ref, wa_ref, wb_ref, b1_ref, xa_ref, xb_ref):
    x = x_ref[...]
    xa_ref[...] = jnp.dot(x, wa_ref[...], preferred_element_type=jnp.float32)
    xb_ref[...] = (jnp.dot(x, wb_ref[...], preferred_element_type=jnp.float32)
                   + b1_ref[...])


_pre_call = pl.pallas_call(
    _pre_body,
    out_shape=(jax.ShapeDtypeStruct((N_NODES_K, LAT), jnp.float32),
               jax.ShapeDtypeStruct((N_NODES_K, LAT), jnp.float32)),
)


LROWS = 640
LTAIL = N_NODES_K - 15 * LROWS


PCH = 80
HW = PCH // 2


def _gather_body(xa_hbm, xb_hbm, row_hbm, col_hbm, g_hbm,
                 xa_sh, ridx, cidx, b0, b1, s0, s1, sw):
    sid = lax.axis_index("s")
    wid = sid * NC + lax.axis_index("c")

    c0 = jnp.minimum(wid * CPW, N_CHUNKS - PCH)

    @pl.when(sid < NS - 1)
    def _():
        pltpu.sync_copy(xa_hbm.at[pl.ds(sid * LROWS, LROWS)],
                        xa_sh.at[pl.ds(sid * LROWS, LROWS)])

    @pl.when(sid == NS - 1)
    def _():
        pltpu.sync_copy(xa_hbm.at[pl.ds((NS - 1) * LROWS, LTAIL)],
                        xa_sh.at[pl.ds((NS - 1) * LROWS, LTAIL)])

    plsc.subcore_barrier()

    bufs = (b0, b1)
    sems = (s0, s1)

    for h in range(2):
        ch = c0 + h * HW
        pltpu.sync_copy(row_hbm.at[pl.ds(ch * CHUNK, HW * CHUNK)], ridx)
        pltpu.sync_copy(col_hbm.at[pl.ds(ch * CHUNK, HW * CHUNK)], cidx)

        def j_body(j, carry):
            adds = []
            for b in range(2):
                i = j * 2 + b
                ri = ridx.at[pl.ds(i * CHUNK, CHUNK)]
                ci = cidx.at[pl.ds(i * CHUNK, CHUNK)]
                pltpu.sync_copy(xa_sh.at[ri], bufs[b])
                adds.append(pltpu.async_copy(xb_hbm.at[ci], bufs[b],
                                             sems[b], add=True))
            adds[0].wait()
            base0 = (ch + j * 2) * CHUNK
            w0 = pltpu.async_copy(b0, g_hbm.at[pl.ds(base0, CHUNK)], sw)
            adds[1].wait()
            pltpu.sync_copy(b1, g_hbm.at[pl.ds(base0 + CHUNK, CHUNK)])
            w0.wait()
            return carry

        lax.fori_loop(0, HW // 2, j_body, 0)


_gather_call = pl.kernel(
    _gather_body,
    out_type=jax.ShapeDtypeStruct((N_EDGES_K, LAT), jnp.float32),
    mesh=_SC_MESH,
    scratch_types=(
        pltpu.VMEM_SHARED((N_NODES_K, LAT), jnp.float32),
        pltpu.VMEM((HW * CHUNK,), jnp.int32),
        pltpu.VMEM((HW * CHUNK,), jnp.int32),
        pltpu.VMEM((CHUNK, LAT), jnp.float32),
        pltpu.VMEM((CHUNK, LAT), jnp.float32),
        pltpu.SemaphoreType.DMA,
        pltpu.SemaphoreType.DMA,
        pltpu.SemaphoreType.DMA,
    ),
)


EBLK = 4000


def _edge_body(g_ref, e_ref, w1c_ref, w2_ref, b2_ref, out_ref):
    e = e_ref[...]
    h = jnp.dot(e.astype(jnp.bfloat16), w1c_ref[...].astype(jnp.bfloat16),
                preferred_element_type=jnp.float32)
    h = jnp.maximum(h + g_ref[...].astype(jnp.float32), 0.0)
    out_ref[...] = (e + jnp.dot(h.astype(jnp.bfloat16),
                                w2_ref[...].astype(jnp.bfloat16),
                                preferred_element_type=jnp.float32)
                    + b2_ref[...])


_edge_call = pl.pallas_call(
    _edge_body,
    grid=(N_EDGES_K // EBLK,),
    in_specs=[
        pl.BlockSpec((EBLK, LAT), lambda i: (i, 0)),
        pl.BlockSpec((EBLK, LAT), lambda i: (i, 0)),
        pl.BlockSpec((LAT, LAT), lambda i: (0, 0)),
        pl.BlockSpec((LAT, LAT), lambda i: (0, 0)),
        pl.BlockSpec((1, LAT), lambda i: (0, 0)),
    ],
    out_specs=pl.BlockSpec((EBLK, LAT), lambda i: (i, 0)),
    out_shape=jax.ShapeDtypeStruct((N_EDGES_K, LAT), jnp.float32),
    compiler_params=pltpu.CompilerParams(
        dimension_semantics=("parallel",)),
)


def _scatter_body(en_hbm, col2_hbm, out_hbm, acc_shared, cidx,
                  d0, d1, zbuf, t0, t1):
    cid = lax.axis_index("c")
    sid = lax.axis_index("s")
    wid = sid * NC + cid

    def zrow(r, carry):
        for j in range(LAT // 16):
            zbuf[r, pl.ds(j * 16, 16)] = jnp.zeros((16,), jnp.float32)
        return carry

    lax.fori_loop(0, ZROWS, zrow, 0)

    def zslice(i, carry):
        j = i * NS + sid

        @pl.when(j < NZ)
        def _():
            pltpu.sync_copy(zbuf, acc_shared.at[pl.ds(j * ZROWS, ZROWS)])

        return carry

    lax.fori_loop(0, ZPS, zslice, 0)

    c0 = jnp.minimum(wid * CPW, N_CHUNKS - CPW)
    off = wid * CPW - c0
    pltpu.sync_copy(col2_hbm.at[pl.ds(c0 * CHUNK, CPW * CHUNK)], cidx)
    plsc.subcore_barrier()

    dbufs = (d0, d1)
    dsems = (t0, t1)

    def j_body(j, carry):
        loads = []
        for b in range(2):
            c = wid * CPW + j * 2 + b
            base = jnp.minimum(c, N_CHUNKS - 1) * CHUNK
            loads.append(pltpu.async_copy(en_hbm.at[pl.ds(base, CHUNK)],
                                          dbufs[b], dsems[b]))
        for b in range(2):
            i = j * 2 + b
            c = wid * CPW + i
            loads[b].wait()

            @pl.when((i < CPW) & (c < N_CHUNKS))
            def _(b=b, i=i):
                pltpu.sync_copy(
                    dbufs[b],
                    acc_shared.at[cidx.at[pl.ds((off + i) * CHUNK, CHUNK)]],
                    add=True)

        return carry

    lax.fori_loop(0, (CPW + 1) // 2, j_body, 0)
    plsc.subcore_barrier()

    def oslice(i, carry):
        j = i * NS + sid

        @pl.when(j < NFULL)
        def _():
            pltpu.sync_copy(acc_shared.at[pl.ds(j * CHUNK, CHUNK)],
                            out_hbm.at[cid, pl.ds(j * CHUNK, CHUNK)])

        @pl.when(j == NFULL)
        def _():
            pltpu.sync_copy(acc_shared.at[pl.ds(NFULL * CHUNK, NTAIL)],
                            out_hbm.at[cid, pl.ds(NFULL * CHUNK, NTAIL)])

        return carry

    lax.fori_loop(0, SPS, oslice, 0)


_scatter_call = pl.kernel(
    _scatter_body,
    out_type=jax.ShapeDtypeStruct((NC, N_NODES_K, LAT), jnp.float32),
    mesh=_SC_MESH,
    scratch_types=(
        pltpu.VMEM_SHARED((N_NODES_K, LAT), jnp.float32),
        pltpu.VMEM((CPW * CHUNK,), jnp.int32),
        pltpu.VMEM((CHUNK, LAT), jnp.float32),
        pltpu.VMEM((CHUNK, LAT), jnp.float32),
        pltpu.VMEM((ZROWS, LAT), jnp.float32),
        pltpu.SemaphoreType.DMA,
        pltpu.SemaphoreType.DMA,
    ),
)


def _node_body(x_ref, agg_ref, w1a_ref, w1b_ref, b1_ref, w2_ref, b2_ref,
               out_ref):
    x = x_ref[...]
    s = agg_ref[0] + agg_ref[1]
    h = (jnp.dot(x, w1a_ref[...], preferred_element_type=jnp.float32)
         + jnp.dot(s, w1b_ref[...], preferred_element_type=jnp.float32)
         + b1_ref[...])
    h = jnp.maximum(h, 0.0)
    out_ref[...] = (x + jnp.dot(h, w2_ref[...],
                                preferred_element_type=jnp.float32)
                    + b2_ref[...])


_node_call = pl.pallas_call(
    _node_body,
    out_shape=jax.ShapeDtypeStruct((N_NODES_K, LAT), jnp.float32),
)


def kernel(x, edge_index, edge_attr, eW1, eb1, eW2, eb2, nW1, nb1, nW2, nb2):
    row2 = edge_index[0]
    col2 = edge_index[1]
    for l in range(2):
        w1a = eW1[l, :LAT]
        w1b = eW1[l, LAT:2 * LAT]
        w1c = eW1[l, 2 * LAT:]
        xa, xb = _pre_call(x, w1a, w1b, eb1[l].reshape(1, LAT))
        g = _gather_call(xa, xb, row2, col2)
        edge_attr = _edge_call(g, edge_attr, w1c, eW2[l],
                               eb2[l].reshape(1, LAT))
        agg2 = _scatter_call(edge_attr, col2)
        x = _node_call(x, agg2, nW1[l, :LAT], nW1[l, LAT:],
                       nb1[l].reshape(1, LAT), nW2[l],
                       nb2[l].reshape(1, LAT))
    return (x, edge_attr)
